# Initial kernel scaffold; baseline (speedup 1.0000x reference)
import jax
import jax.numpy as jnp
from jax.experimental import pallas as pl


def kernel(idx, wte, wpe, ln_w, ln_b):
    raise NotImplementedError("write your pallas kernel here")


# speedup vs baseline: 1.1209x; 1.1209x over previous
import functools

import jax
import jax.numpy as jnp
from jax import lax
from jax.experimental import pallas as pl
from jax.experimental.pallas import tpu as pltpu
from jax.experimental.pallas import tpu_sc as plsc

NC = 2
NS = 16
NW = NC * NS
L = 16
D = 128
ND = D // L
C = 128
EPS = 1e-5


def _rsqrt(v):
    i = plsc.bitcast(v, jnp.int32)
    i = 0x5F3759DF - lax.shift_right_arithmetic(i, 1)
    y = plsc.bitcast(i, jnp.float32)
    for _ in range(3):
        y = y * (1.5 - 0.5 * v * y * y)
    return y


def _body(T, idx_hbm, wte_hbm, wpe_hbm, lnw_hbm, lnb_hbm, out_hbm,
          idx_v, tok_v, wpe_v, out_v, lnw_v, lnb_v, sem):
    w = lax.axis_index("s") * NC + lax.axis_index("c")

    pltpu.sync_copy(lnw_hbm, lnw_v)
    pltpu.sync_copy(lnb_hbm, lnb_v)
    lnw = [lnw_v[pl.ds(L * d, L)] for d in range(ND)]
    lnb = [lnb_v[pl.ds(L * d, L)] for d in range(ND)]

    iot = lax.iota(jnp.int32, L)
    perms = [iot ^ k for k in (1, 2, 4, 8)]

    def lane_sum(x):
        for p in perms:
            x = x + x.at[p].get(mode="promise_in_bounds")
        return x

    def row(r, carry):
        xs = []
        acc = None
        sq = None
        for d in range(ND):
            t = tok_v[r, pl.ds(L * d, L)] + wpe_v[r, pl.ds(L * d, L)]
            xs.append(t)
            acc = t if acc is None else acc + t
            sq = t * t if sq is None else sq + t * t
        mean = lane_sum(acc) * (1.0 / D)
        var = lane_sum(sq) * (1.0 / D) - mean * mean + EPS
        rv = _rsqrt(var)
        for d in range(ND):
            out_v[r, pl.ds(L * d, L)] = (xs[d] - mean) * rv * lnw[d] + lnb[d]
        return carry

    for c in range(T // C):
        pltpu.sync_copy(idx_hbm.at[w, pl.ds(c * C, C)], idx_v)
        gat = pltpu.async_copy(wte_hbm.at[idx_v], tok_v, sem)
        pltpu.sync_copy(wpe_hbm.at[pl.ds(c * C, C)], wpe_v)
        gat.wait()
        lax.fori_loop(0, C, row, 0)
        pltpu.sync_copy(out_v, out_hbm.at[w, pl.ds(c * C, C)])


def kernel(idx, wte, wpe, ln_w, ln_b):
    B, T = idx.shape
    _, d_model = wte.shape
    assert d_model == D and B == NW and T % C == 0

    mesh = plsc.VectorSubcoreMesh(core_axis_name="c", subcore_axis_name="s")
    k = pl.kernel(
        functools.partial(_body, T),
        out_type=jax.ShapeDtypeStruct((B, T, D), jnp.float32),
        mesh=mesh,
        compiler_params=pltpu.CompilerParams(needs_layout_passes=False),
        scratch_types=[
            pltpu.VMEM((C,), jnp.int32),
            pltpu.VMEM((C, D), jnp.float32),
            pltpu.VMEM((C, D), jnp.float32),
            pltpu.VMEM((C, D), jnp.float32),
            pltpu.VMEM((D,), jnp.float32),
            pltpu.VMEM((D,), jnp.float32),
            pltpu.SemaphoreType.DMA,
        ],
    )
    return k(idx, wte, wpe, ln_w, ln_b)


# speedup vs baseline: 1.5772x; 1.4071x over previous
import functools

import jax
import jax.numpy as jnp
from jax import lax
from jax.experimental import pallas as pl
from jax.experimental.pallas import tpu as pltpu
from jax.experimental.pallas import tpu_sc as plsc

NC = 2
NS = 16
NW = NC * NS
L = 16
D = 128
ND = D // L
C = 128
NBUF = 2
EPS = 1e-5


def _rsqrt(v):
    i = plsc.bitcast(v, jnp.int32)
    i = 0x5F3759DF - lax.shift_right_arithmetic(i, 1)
    y = plsc.bitcast(i, jnp.float32)
    for _ in range(3):
        y = y * (1.5 - 0.5 * v * y * y)
    return y


def _body(T, idx_hbm, wte_hbm, wpe_hbm, lnw_hbm, lnb_hbm, out_hbm,
          idx_v, tok_v, wpe_v, out_v, lnw_v, lnb_v,
          gsems, wsems, osems):
    w = lax.axis_index("s") * NC + lax.axis_index("c")
    n_chunks = T // C

    pltpu.sync_copy(lnw_hbm, lnw_v)
    pltpu.sync_copy(lnb_hbm, lnb_v)
    pltpu.sync_copy(idx_hbm.at[w], idx_v)
    lnw = [lnw_v[pl.ds(L * d, L)] for d in range(ND)]
    lnb = [lnb_v[pl.ds(L * d, L)] for d in range(ND)]

    iot = lax.iota(jnp.int32, L)
    perms = [iot ^ k for k in (1, 2, 4, 8)]

    def lane_sum(x):
        for p in perms:
            x = x + x.at[p].get(mode="promise_in_bounds")
        return x

    def make_row(s):
        def row(r, carry):
            xs = []
            acc = None
            sq = None
            for d in range(ND):
                t = tok_v[s, r, pl.ds(L * d, L)] + wpe_v[s, r, pl.ds(L * d, L)]
                xs.append(t)
                acc = t if acc is None else acc + t
                sq = t * t if sq is None else sq + t * t
            mean = lane_sum(acc) * (1.0 / D)
            var = lane_sum(sq) * (1.0 / D) - mean * mean + EPS
            rv = _rsqrt(var)
            for d in range(ND):
                out_v[s, r, pl.ds(L * d, L)] = (xs[d] - mean) * rv * lnw[d] + lnb[d]
            return carry
        return row

    def start_fetch(c):
        s = c % NBUF
        g = pltpu.async_copy(wte_hbm.at[idx_v.at[pl.ds(c * C, C)]],
                             tok_v.at[s], gsems[s])
        p = pltpu.async_copy(wpe_hbm.at[pl.ds(c * C, C)], wpe_v.at[s],
                             wsems[s])
        return g, p

    inflight = {}
    out_inflight = {}
    inflight[0] = start_fetch(0)
    for c in range(n_chunks):
        s = c % NBUF
        if c + 1 < n_chunks:
            inflight[c + 1] = start_fetch(c + 1)
        g, p = inflight.pop(c)
        g.wait()
        p.wait()
        if c - NBUF in out_inflight:
            out_inflight.pop(c - NBUF).wait()
        lax.fori_loop(0, C, make_row(s), 0)
        out_inflight[c] = pltpu.async_copy(
            out_v.at[s], out_hbm.at[w, pl.ds(c * C, C)], osems[s])
    for c in sorted(out_inflight):
        out_inflight.pop(c).wait()


def kernel(idx, wte, wpe, ln_w, ln_b):
    B, T = idx.shape
    _, d_model = wte.shape
    assert d_model == D and B == NW and T % C == 0

    mesh = plsc.VectorSubcoreMesh(core_axis_name="c", subcore_axis_name="s")
    k = pl.kernel(
        functools.partial(_body, T),
        out_type=jax.ShapeDtypeStruct((B, T, D), jnp.float32),
        mesh=mesh,
        compiler_params=pltpu.CompilerParams(needs_layout_passes=False),
        scratch_types=[
            pltpu.VMEM((T,), jnp.int32),
            pltpu.VMEM((NBUF, C, D), jnp.float32),
            pltpu.VMEM((NBUF, C, D), jnp.float32),
            pltpu.VMEM((NBUF, C, D), jnp.float32),
            pltpu.VMEM((D,), jnp.float32),
            pltpu.VMEM((D,), jnp.float32),
            [pltpu.SemaphoreType.DMA] * NBUF,
            [pltpu.SemaphoreType.DMA] * NBUF,
            [pltpu.SemaphoreType.DMA] * NBUF,
        ],
    )
    return k(idx, wte, wpe, ln_w, ln_b)
